# baseline (speedup 1.0000x reference)
import functools

import jax
import jax.numpy as jnp
from jax import lax
from jax.experimental import pallas as pl
from jax.experimental.pallas import tpu as pltpu
from jax.experimental.pallas import tpu_sc as plsc

_NC, _NS, _L = 2, 16, 16
_NW = _NC * _NS
_C = 16


def _build(N, K, D):
    n_chunks = N // _C
    assert n_chunks * _C == N and K == _L
    nt = -(-n_chunks // _NW)
    assert nt % 2 == 0 and nt >= 6
    KD = K * D
    nd = D // _L

    mesh = plsc.VectorSubcoreMesh(
        core_axis_name="c", subcore_axis_name="s",
        num_cores=_NC, num_subcores=_NS)

    @functools.partial(
        pl.kernel,
        out_type=jax.ShapeDtypeStruct((N, KD), jnp.float32),
        mesh=mesh,
        scratch_types=[
            pltpu.VMEM((_C, K), jnp.int32),
            pltpu.VMEM((_C, K), jnp.int32),
            pltpu.VMEM((_C,), jnp.int32),
            pltpu.VMEM((_C,), jnp.int32),
            pltpu.VMEM((2, D), jnp.float32),
            pltpu.VMEM((_C, KD), jnp.float32),
            pltpu.VMEM((_C, KD), jnp.float32),
            pltpu.SemaphoreType.DMA,
            pltpu.SemaphoreType.DMA,
            pltpu.SemaphoreType.DMA,
            pltpu.SemaphoreType.DMA,
        ],
    )
    def run(src_hbm, dst_hbm, w_hbm, out_hbm,
            szv0, szv1, dzv0, dzv1, wv, ob0, ob1, si0, si1, so0, so1):
        wid = lax.axis_index("s") * _NC + lax.axis_index("c")
        pltpu.sync_copy(w_hbm, wv)
        w0 = [wv[0, pl.ds(c * _L, _L)] for c in range(nd)]
        w1 = [wv[1, pl.ds(c * _L, _L)] for c in range(nd)]
        dw = [w1[c] - w0[c] for c in range(nd)]
        ones = jnp.full((_L,), 1.0, jnp.float32)
        zeros = jnp.full((_L,), 0.0, jnp.float32)
        bufs = ((szv0, dzv0, ob0, si0, so0), (szv1, dzv1, ob1, si1, so1))

        def r0_of(t):
            cid = wid + t * _NW
            return jnp.where(cid < n_chunks, cid, cid - _NW) * _C

        def issue_in(t, p):
            szv, dzv, _, si, _ = bufs[p]
            r0 = r0_of(t)
            pltpu.async_copy(src_hbm.at[pl.ds(r0, _C)], szv, si)
            pltpu.async_copy(dst_hbm.at[pl.ds(r0, _C)], dzv, si)

        def wait_in(p):
            szv, dzv, _, si, _ = bufs[p]
            pltpu.make_async_copy(src_hbm.at[pl.ds(0, _C)], szv, si).wait()
            pltpu.make_async_copy(dst_hbm.at[pl.ds(0, _C)], dzv, si).wait()

        def issue_out(t, p):
            _, _, ob, _, so = bufs[p]
            pltpu.async_copy(ob, out_hbm.at[pl.ds(r0_of(t), _C)], so)

        def wait_out(p):
            _, _, ob, _, so = bufs[p]
            pltpu.make_async_copy(ob, out_hbm.at[pl.ds(0, _C)], so).wait()

        def compute(p):
            szv, dzv, ob, _, _ = bufs[p]
            dvec = dzv[...]
            for r in range(_C):
                srow = szv[r, :]
                mask = srow == jnp.full((_L,), dvec[r], jnp.int32)
                mf = jnp.where(mask, ones, zeros)
                for j in range(K):
                    m = mf[j]
                    for c in range(nd):
                        ob[r, pl.ds(j * D + c * _L, _L)] = w0[c] + m * dw[c]

        issue_in(0, 0)
        issue_in(1, 1)
        for p in (0, 1):
            wait_in(p)
            compute(p)
            issue_out(p, p)
            issue_in(p + 2, p)

        def super_body(s, carry):
            for p in (0, 1):
                t = 2 * s + p
                wait_in(p)
                wait_out(p)
                compute(p)
                issue_out(t, p)
                issue_in(t + 2, p)
            return carry

        lax.fori_loop(1, nt // 2 - 1, super_body, 0)

        for p in (0, 1):
            wait_in(p)
            wait_out(p)
            compute(p)
            issue_out(nt - 2 + p, p)
        wait_out(0)
        wait_out(1)

    return run


_TC_ROWS = 38912
_TC_BR = 256


def _tc_select2d(src_ref, dst_ref, pd_ref, w0_ref, out_ref):
    zf = (src_ref[...] == dst_ref[...]).astype(jnp.float32)
    out_ref[...] = jnp.dot(zf, pd_ref[...],
                           preferred_element_type=jnp.float32) + w0_ref[...]


def kernel(src_z, dst_z, k, e1_weight):
    N, K = src_z.shape
    D = e1_weight.shape[1]
    KD = K * D
    H = _TC_ROWS
    R = N - H
    BR = _TC_BR

    dw_row = e1_weight[1] - e1_weight[0]
    pd = (jnp.eye(K, dtype=jnp.float32)[:, :, None]
          * dw_row[None, None, :]).reshape(K, KD)
    w0kd = jnp.tile(e1_weight[0], K)[None, :]
    dst2 = jnp.broadcast_to(dst_z[:H, None], (H, K))
    tc2d = pl.pallas_call(
        _tc_select2d,
        grid=(H // BR,),
        in_specs=[
            pl.BlockSpec((BR, K), lambda i: (i, 0)),
            pl.BlockSpec((BR, K), lambda i: (i, 0)),
            pl.BlockSpec((K, KD), lambda i: (0, 0)),
            pl.BlockSpec((1, KD), lambda i: (0, 0)),
        ],
        out_specs=pl.BlockSpec((BR, KD), lambda i: (i, 0)),
        out_shape=jax.ShapeDtypeStruct((H, KD), jnp.float32),
    )(src_z[:H], dst2, pd, w0kd)

    run = _build(R, K, D)
    sc2d = run(src_z[H:], dst_z[H:], e1_weight)

    return jnp.concatenate([tc2d, sc2d], axis=0).reshape(N, K, D)


# speedup vs baseline: 1.2110x; 1.2110x over previous
import functools

import jax
import jax.numpy as jnp
from jax import lax
from jax.experimental import pallas as pl
from jax.experimental.pallas import tpu as pltpu
from jax.experimental.pallas import tpu_sc as plsc

_NC, _NS, _L = 2, 16, 16
_NW = _NC * _NS
_C = 16


def _build(N, K, D):
    n_chunks = N // _C
    assert n_chunks * _C == N and K == _L
    nt = -(-n_chunks // _NW)
    assert nt % 2 == 0 and nt >= 6
    KD = K * D
    nd = D // _L

    mesh = plsc.VectorSubcoreMesh(
        core_axis_name="c", subcore_axis_name="s",
        num_cores=_NC, num_subcores=_NS)

    @functools.partial(
        pl.kernel,
        out_type=jax.ShapeDtypeStruct((N, KD), jnp.float32),
        mesh=mesh,
        scratch_types=[
            pltpu.VMEM((_C, K), jnp.int32),
            pltpu.VMEM((_C, K), jnp.int32),
            pltpu.VMEM((_C,), jnp.int32),
            pltpu.VMEM((_C,), jnp.int32),
            pltpu.VMEM((2, D), jnp.float32),
            pltpu.VMEM((_C, KD), jnp.float32),
            pltpu.VMEM((_C, KD), jnp.float32),
            pltpu.SemaphoreType.DMA,
            pltpu.SemaphoreType.DMA,
            pltpu.SemaphoreType.DMA,
            pltpu.SemaphoreType.DMA,
        ],
    )
    def run(src_hbm, dst_hbm, w_hbm, out_hbm,
            szv0, szv1, dzv0, dzv1, wv, ob0, ob1, si0, si1, so0, so1):
        wid = lax.axis_index("s") * _NC + lax.axis_index("c")
        pltpu.sync_copy(w_hbm, wv)
        w0 = [wv[0, pl.ds(c * _L, _L)] for c in range(nd)]
        w1 = [wv[1, pl.ds(c * _L, _L)] for c in range(nd)]
        dw = [w1[c] - w0[c] for c in range(nd)]
        ones = jnp.full((_L,), 1.0, jnp.float32)
        zeros = jnp.full((_L,), 0.0, jnp.float32)
        bufs = ((szv0, dzv0, ob0, si0, so0), (szv1, dzv1, ob1, si1, so1))

        def r0_of(t):
            cid = wid + t * _NW
            return jnp.where(cid < n_chunks, cid, cid - _NW) * _C

        def issue_in(t, p):
            szv, dzv, _, si, _ = bufs[p]
            r0 = r0_of(t)
            pltpu.async_copy(src_hbm.at[pl.ds(r0, _C)], szv, si)
            pltpu.async_copy(dst_hbm.at[pl.ds(r0, _C)], dzv, si)

        def wait_in(p):
            szv, dzv, _, si, _ = bufs[p]
            pltpu.make_async_copy(src_hbm.at[pl.ds(0, _C)], szv, si).wait()
            pltpu.make_async_copy(dst_hbm.at[pl.ds(0, _C)], dzv, si).wait()

        def issue_out_half(t, p, h):
            _, _, ob, _, so = bufs[p]
            hh = _C // 2
            pltpu.async_copy(ob.at[pl.ds(h * hh, hh)],
                             out_hbm.at[pl.ds(r0_of(t) + h * hh, hh)], so)

        def wait_out(p):
            _, _, ob, _, so = bufs[p]
            pltpu.make_async_copy(ob, out_hbm.at[pl.ds(0, _C)], so).wait()

        def compute(p, t):
            szv, dzv, ob, _, _ = bufs[p]
            dvec = dzv[...]
            for r in range(_C):
                srow = szv[r, :]
                mask = srow == jnp.full((_L,), dvec[r], jnp.int32)
                mf = jnp.where(mask, ones, zeros)
                for j in range(K):
                    m = mf[j]
                    for c in range(nd):
                        ob[r, pl.ds(j * D + c * _L, _L)] = w0[c] + m * dw[c]
                if r == _C // 2 - 1:
                    issue_out_half(t, p, 0)
            issue_out_half(t, p, 1)

        issue_in(0, 0)
        issue_in(1, 1)
        for p in (0, 1):
            wait_in(p)
            compute(p, p)
            issue_in(p + 2, p)

        def super_body(s, carry):
            for p in (0, 1):
                t = 2 * s + p
                wait_in(p)
                wait_out(p)
                compute(p, t)
                issue_in(t + 2, p)
            return carry

        lax.fori_loop(1, nt // 2 - 1, super_body, 0)

        for p in (0, 1):
            wait_in(p)
            wait_out(p)
            compute(p, nt - 2 + p)
        wait_out(0)
        wait_out(1)

    return run


def kernel(src_z, dst_z, k, e1_weight):
    N, K = src_z.shape
    D = e1_weight.shape[1]
    run = _build(N, K, D)
    out = run(src_z, dst_z, e1_weight)
    return out.reshape(N, K, D)


# speedup vs baseline: 1.2229x; 1.0098x over previous
import functools

import jax
import jax.numpy as jnp
from jax import lax
from jax.experimental import pallas as pl
from jax.experimental.pallas import tpu as pltpu
from jax.experimental.pallas import tpu_sc as plsc

_NC, _NS, _L = 2, 16, 16
_NW = _NC * _NS
_C = 16


def _build(N, K, D):
    n_chunks = N // _C
    assert n_chunks * _C == N and K == _L
    nt = -(-n_chunks // _NW)
    assert nt % 2 == 0 and nt >= 6
    KD = K * D
    nd = D // _L

    mesh = plsc.VectorSubcoreMesh(
        core_axis_name="c", subcore_axis_name="s",
        num_cores=_NC, num_subcores=_NS)

    @functools.partial(
        pl.kernel,
        out_type=jax.ShapeDtypeStruct((N, KD), jnp.float32),
        mesh=mesh,
        scratch_types=[
            pltpu.VMEM((_C, K), jnp.int32),
            pltpu.VMEM((_C, K), jnp.int32),
            pltpu.VMEM((_C,), jnp.int32),
            pltpu.VMEM((_C,), jnp.int32),
            pltpu.VMEM((2, D), jnp.float32),
            pltpu.VMEM((_C, KD), jnp.float32),
            pltpu.VMEM((_C, KD), jnp.float32),
            pltpu.SemaphoreType.DMA,
            pltpu.SemaphoreType.DMA,
            pltpu.SemaphoreType.DMA,
            pltpu.SemaphoreType.DMA,
        ],
    )
    def run(src_hbm, dst_hbm, w_hbm, out_hbm,
            szv0, szv1, dzv0, dzv1, wv, ob0, ob1, si0, si1, so0, so1):
        wid = lax.axis_index("s") * _NC + lax.axis_index("c")
        pltpu.sync_copy(w_hbm, wv)
        w0 = [wv[0, pl.ds(c * _L, _L)] for c in range(nd)]
        w1 = [wv[1, pl.ds(c * _L, _L)] for c in range(nd)]
        dw = [w1[c] - w0[c] for c in range(nd)]
        ones = jnp.full((_L,), 1.0, jnp.float32)
        zeros = jnp.full((_L,), 0.0, jnp.float32)
        bufs = ((szv0, dzv0, ob0, si0, so0), (szv1, dzv1, ob1, si1, so1))

        def r0_of(t):
            cid = wid + t * _NW
            return jnp.where(cid < n_chunks, cid, cid - _NW) * _C

        def issue_in(t, p):
            szv, dzv, _, si, _ = bufs[p]
            r0 = r0_of(t)
            pltpu.async_copy(src_hbm.at[pl.ds(r0, _C)], szv, si)
            pltpu.async_copy(dst_hbm.at[pl.ds(r0, _C)], dzv, si)

        def wait_in(p):
            szv, dzv, _, si, _ = bufs[p]
            pltpu.make_async_copy(src_hbm.at[pl.ds(0, _C)], szv, si).wait()
            pltpu.make_async_copy(dst_hbm.at[pl.ds(0, _C)], dzv, si).wait()

        def issue_out(t, p):
            _, _, ob, _, so = bufs[p]
            pltpu.async_copy(ob, out_hbm.at[pl.ds(r0_of(t), _C)], so)

        def wait_out(p):
            _, _, ob, _, so = bufs[p]
            pltpu.make_async_copy(ob, out_hbm.at[pl.ds(0, _C)], so).wait()

        def compute(p):
            szv, dzv, ob, _, _ = bufs[p]
            dvec = dzv[...]
            for r in range(_C):
                srow = szv[r, :]
                mask = srow == jnp.full((_L,), dvec[r], jnp.int32)
                mf = jnp.where(mask, ones, zeros)
                for j in range(K):
                    m = mf[j]
                    for c in range(nd):
                        ob[r, pl.ds(j * D + c * _L, _L)] = w0[c] + m * dw[c]

        issue_in(0, 0)
        issue_in(1, 1)
        for p in (0, 1):
            wait_in(p)
            compute(p)
            issue_out(p, p)
            issue_in(p + 2, p)

        def super_body(s, carry):
            for p in (0, 1):
                t = 2 * s + p
                wait_in(p)
                wait_out(p)
                compute(p)
                issue_out(t, p)
                issue_in(t + 2, p)
            return carry

        lax.fori_loop(1, nt // 2 - 1, super_body, 0)

        for p in (0, 1):
            wait_in(p)
            wait_out(p)
            compute(p)
            issue_out(nt - 2 + p, p)
        wait_out(0)
        wait_out(1)

    return run


def kernel(src_z, dst_z, k, e1_weight):
    N, K = src_z.shape
    D = e1_weight.shape[1]
    run = _build(N, K, D)
    out = run(src_z, dst_z, e1_weight)
    return out.reshape(N, K, D)
